# baseline (speedup 1.0000x reference)
import functools
import math

import jax
import jax.numpy as jnp
from jax import lax
from jax.experimental import pallas as pl
from jax.experimental.pallas import tpu as pltpu

NS = 48
SH = 9
N_NODES = 10000
N_EDGES = 320000

EDGE_BLK = 1000

_MM_PREC = lax.Precision.HIGHEST


def _dot(a, b):
    return jnp.dot(a, b, precision=_MM_PREC, preferred_element_type=jnp.float32)


def _node_embed_body(x_ref, sig_ref, wx_ref, ws_ref, b_ref, o_ref):
    acc = _dot(x_ref[...], wx_ref[...]) + _dot(sig_ref[...], ws_ref[...])
    o_ref[...] = jnp.maximum(acc + b_ref[...], 0.0)


def _node_embed(x, sig, W_node, b_node):
    n = x.shape[0]
    return pl.pallas_call(
        _node_embed_body,
        out_shape=jax.ShapeDtypeStruct((n, NS), jnp.float32),
    )(x, sig, W_node[:, :16].T, W_node[:, 16:].T, b_node.reshape(1, NS))


def _edge_prep_body(ea_ref, ps_ref, pd_ref, sg_ref, w1a_ref, w1b_ref, w1c_ref,
                    b1_ref, w2_ref, b2_ref, e_ref, dv_ref):
    ps = ps_ref[...]
    pd = pd_ref[...]
    vec = pd[:, 0:3] - ps[:, 0:3]
    d2 = jnp.sum(vec * vec, axis=1, keepdims=True)
    d = jnp.sqrt(d2)
    offs = lax.broadcasted_iota(jnp.int32, (1, 32), 1).astype(jnp.float32) * (5.0 / 31.0)
    coeff = -0.5 / (5.0 / 31.0) ** 2
    smear = jnp.exp(coeff * (d - offs) ** 2)
    acc = (_dot(ea_ref[...], w1a_ref[...])
           + _dot(sg_ref[...], w1b_ref[...])
           + _dot(smear, w1c_ref[...]))
    h1 = jnp.maximum(acc + b1_ref[...], 0.0)
    e_ref[...] = _dot(h1, w2_ref[...]) + b2_ref[...]
    dvec = vec / jnp.maximum(d, 1e-8)
    blk = dvec.shape[0]
    dv_ref[...] = jnp.concatenate(
        [dvec, jnp.zeros((blk, 13), jnp.float32)], axis=1)


def _edge_prep(edge_attr8, psrc, pdst, sigsrc, We1, be1, We2, be2):
    e_count = edge_attr8.shape[0]
    grid = (e_count // EDGE_BLK,)
    eb = lambda w: pl.BlockSpec((EDGE_BLK, w), lambda i: (i, 0))
    wb = lambda s: pl.BlockSpec(s, lambda i: (0, 0))
    return pl.pallas_call(
        _edge_prep_body,
        grid=grid,
        in_specs=[eb(8), eb(16), eb(16), eb(32),
                  wb((8, NS)), wb((32, NS)), wb((32, NS)), wb((1, NS)),
                  wb((NS, NS)), wb((1, NS))],
        out_specs=[eb(NS), eb(16)],
        out_shape=[jax.ShapeDtypeStruct((e_count, NS), jnp.float32),
                   jax.ShapeDtypeStruct((e_count, 16), jnp.float32)],
    )(edge_attr8, psrc, pdst, sigsrc,
      jnp.concatenate([We1[:, :5].T, jnp.zeros((3, NS), jnp.float32)], axis=0),
      We1[:, 5:37].T, We1[:, 37:69].T, be1.reshape(1, NS),
      We2.T, be2.reshape(1, NS))


def _conv_body(e_ref, hs_ref, hd_ref, dv_ref, w1_ref, b1_ref, w2_ref, b2_ref,
               s_ref, o_ref):
    hs = hs_ref[...]
    feat = jnp.concatenate([e_ref[...], hs, hd_ref[...]], axis=1)
    a = jnp.maximum(_dot(feat, w1_ref[...]) + b1_ref[...], 0.0)
    w2 = _dot(a, w2_ref[...]) + b2_ref[...]
    dv = dv_ref[...]
    dx, dy, dz = dv[:, 0:1], dv[:, 1:2], dv[:, 2:3]
    s3 = math.sqrt(3.0)
    s15 = math.sqrt(15.0)
    s5 = math.sqrt(5.0)
    shv = [jnp.ones_like(dx), s3 * dx, s3 * dy, s3 * dz,
           s15 * dx * dy, s15 * dy * dz,
           (s5 / 2.0) * (3.0 * dz * dz - 1.0), s15 * dx * dz,
           (s15 / 2.0) * (dx * dx - dy * dy)]
    blk = w2.shape[0]
    lane = lax.broadcasted_iota(jnp.int32, (blk, NS * SH), 1)
    jmod = lane - (lane // SH) * SH
    shexp = jnp.zeros_like(w2)
    for j in range(SH):
        shexp = jnp.where(jmod == j, shv[j], shexp)
    t = _dot(w2 * shexp, s_ref[...])
    o_ref[...] = hs * t


def _conv_layer(e, hsrc, hdst, dvec, Wc1, bc1, Wc2, bc2, sel):
    e_count = e.shape[0]
    grid = (e_count // EDGE_BLK,)
    eb = lambda w: pl.BlockSpec((EDGE_BLK, w), lambda i: (i, 0))
    wb = lambda s: pl.BlockSpec(s, lambda i: (0, 0))
    return pl.pallas_call(
        _conv_body,
        grid=grid,
        in_specs=[eb(NS), eb(NS), eb(NS), eb(16),
                  wb((3 * NS, 3 * NS)), wb((1, 3 * NS)),
                  wb((3 * NS, NS * SH)), wb((1, NS * SH)),
                  wb((NS * SH, NS))],
        out_specs=eb(NS),
        out_shape=jax.ShapeDtypeStruct((e_count, NS), jnp.float32),
    )(e, hsrc, hdst, dvec, Wc1.T, bc1.reshape(1, -1), Wc2.T,
      bc2.reshape(1, -1), sel)


def _update_body(h_ref, acc_ref, o_ref):
    acc = acc_ref[...]
    deg = jnp.maximum(acc[:, NS:NS + 1], 1.0)
    o_ref[...] = h_ref[...] + acc[:, :NS] / deg


def _h_update(h, acc):
    n = h.shape[0]
    return pl.pallas_call(
        _update_body,
        out_shape=jax.ShapeDtypeStruct((n, NS), jnp.float32),
    )(h, acc)


def _gather_rows(table, idx):
    return jnp.take(table, idx, axis=0)


def _scatter_acc(msg, dst, n, with_ones):
    ones = jnp.ones((msg.shape[0], 1), jnp.float32)
    pad = jnp.concatenate([msg, ones, jnp.zeros((msg.shape[0], 15), jnp.float32)],
                          axis=1)
    return jax.ops.segment_sum(pad, dst, num_segments=n)


def kernel(x, pos, node_sigma_emb, edge_attr, W_node, b_node, We1, be1, We2,
           be2, Wc1_0, bc1_0, Wc2_0, bc2_0, Wc1_1, bc1_1, Wc2_1, bc2_1,
           edge_index):
    n = x.shape[0]
    src = edge_index[0]
    dst = edge_index[1]

    h = _node_embed(x, node_sigma_emb, W_node, b_node)

    pos16 = jnp.concatenate([pos, jnp.zeros((n, 13), jnp.float32)], axis=1)
    psrc = _gather_rows(pos16, src)
    pdst = _gather_rows(pos16, dst)
    sigsrc = _gather_rows(node_sigma_emb, src)
    ea8 = jnp.concatenate(
        [edge_attr, jnp.zeros((edge_attr.shape[0], 3), jnp.float32)], axis=1)

    e, dvec = _edge_prep(ea8, psrc, pdst, sigsrc, We1, be1, We2, be2)

    ii = jnp.arange(NS * SH) // SH
    sel = (ii[:, None] == jnp.arange(NS)[None, :]).astype(jnp.float32)

    for (Wc1, bc1, Wc2, bc2) in ((Wc1_0, bc1_0, Wc2_0, bc2_0),
                                 (Wc1_1, bc1_1, Wc2_1, bc2_1)):
        hsrc = _gather_rows(h, src)
        hdst = _gather_rows(h, dst)
        msg = _conv_layer(e, hsrc, hdst, dvec, Wc1, bc1, Wc2, bc2, sel)
        acc = _scatter_acc(msg, dst, n, True)
        h = _h_update(h, acc)
    return h


# speedup vs baseline: 1.3639x; 1.3639x over previous
import functools
import math

import jax
import jax.numpy as jnp
from jax import lax
from jax.experimental import pallas as pl
from jax.experimental.pallas import tpu as pltpu

NS = 48
SH = 9
N_NODES = 10000
N_EDGES = 320000

EDGE_BLK = 1000

_MM_PREC = lax.Precision.DEFAULT


def _dot(a, b):
    return jnp.dot(a, b, precision=_MM_PREC, preferred_element_type=jnp.float32)


def _node_embed_body(x_ref, sig_ref, wx_ref, ws_ref, b_ref, o_ref):
    acc = _dot(x_ref[...], wx_ref[...]) + _dot(sig_ref[...], ws_ref[...])
    o_ref[...] = jnp.maximum(acc + b_ref[...], 0.0)


def _node_embed(x, sig, W_node, b_node):
    n = x.shape[0]
    return pl.pallas_call(
        _node_embed_body,
        out_shape=jax.ShapeDtypeStruct((n, NS), jnp.float32),
    )(x, sig, W_node[:, :16].T, W_node[:, 16:].T, b_node.reshape(1, NS))


def _edge_prep_body(ea_ref, ps_ref, pd_ref, sg_ref, w1a_ref, w1b_ref, w1c_ref,
                    b1_ref, w2_ref, b2_ref, e_ref, dv_ref):
    ps = ps_ref[...]
    pd = pd_ref[...]
    vec = pd[:, 0:3] - ps[:, 0:3]
    d2 = jnp.sum(vec * vec, axis=1, keepdims=True)
    d = jnp.sqrt(d2)
    offs = lax.broadcasted_iota(jnp.int32, (1, 32), 1).astype(jnp.float32) * (5.0 / 31.0)
    coeff = -0.5 / (5.0 / 31.0) ** 2
    smear = jnp.exp(coeff * (d - offs) ** 2)
    acc = (_dot(ea_ref[...], w1a_ref[...])
           + _dot(sg_ref[...], w1b_ref[...])
           + _dot(smear, w1c_ref[...]))
    h1 = jnp.maximum(acc + b1_ref[...], 0.0)
    e_ref[...] = _dot(h1, w2_ref[...]) + b2_ref[...]
    dvec = vec / jnp.maximum(d, 1e-8)
    blk = dvec.shape[0]
    dv_ref[...] = jnp.concatenate(
        [dvec, jnp.zeros((blk, 13), jnp.float32)], axis=1)


def _edge_prep(edge_attr8, psrc, pdst, sigsrc, We1, be1, We2, be2):
    e_count = edge_attr8.shape[0]
    grid = (e_count // EDGE_BLK,)
    eb = lambda w: pl.BlockSpec((EDGE_BLK, w), lambda i: (i, 0))
    wb = lambda s: pl.BlockSpec(s, lambda i: (0, 0))
    return pl.pallas_call(
        _edge_prep_body,
        grid=grid,
        in_specs=[eb(8), eb(16), eb(16), eb(32),
                  wb((8, NS)), wb((32, NS)), wb((32, NS)), wb((1, NS)),
                  wb((NS, NS)), wb((1, NS))],
        out_specs=[eb(NS), eb(16)],
        out_shape=[jax.ShapeDtypeStruct((e_count, NS), jnp.float32),
                   jax.ShapeDtypeStruct((e_count, 16), jnp.float32)],
        compiler_params=pltpu.CompilerParams(
            dimension_semantics=("parallel",)),
    )(edge_attr8, psrc, pdst, sigsrc,
      jnp.concatenate([We1[:, :5].T, jnp.zeros((3, NS), jnp.float32)], axis=0),
      We1[:, 5:37].T, We1[:, 37:69].T, be1.reshape(1, NS),
      We2.T, be2.reshape(1, NS))


def _conv_body(e_ref, hs_ref, hd_ref, dv_ref, w1_ref, b1_ref, w2_ref, b2_ref,
               s_ref, o_ref):
    hs = hs_ref[...]
    feat = jnp.concatenate([e_ref[...], hs, hd_ref[...]], axis=1)
    a = jnp.maximum(_dot(feat, w1_ref[...]) + b1_ref[...], 0.0)
    w2 = _dot(a, w2_ref[...]) + b2_ref[...]
    dv = dv_ref[...]
    dx, dy, dz = dv[:, 0:1], dv[:, 1:2], dv[:, 2:3]
    s3 = math.sqrt(3.0)
    s15 = math.sqrt(15.0)
    s5 = math.sqrt(5.0)
    shv = [jnp.ones_like(dx), s3 * dx, s3 * dy, s3 * dz,
           s15 * dx * dy, s15 * dy * dz,
           (s5 / 2.0) * (3.0 * dz * dz - 1.0), s15 * dx * dz,
           (s15 / 2.0) * (dx * dx - dy * dy)]
    blk = w2.shape[0]
    lane = lax.broadcasted_iota(jnp.int32, (blk, NS * SH), 1)
    jmod = lane - (lane // SH) * SH
    shexp = jnp.zeros_like(w2)
    for j in range(SH):
        shexp = jnp.where(jmod == j, shv[j], shexp)
    t = _dot(w2 * shexp, s_ref[...])
    o_ref[...] = hs * t


def _conv_layer(e, hsrc, hdst, dvec, Wc1, bc1, Wc2, bc2, sel):
    e_count = e.shape[0]
    grid = (e_count // EDGE_BLK,)
    eb = lambda w: pl.BlockSpec((EDGE_BLK, w), lambda i: (i, 0))
    wb = lambda s: pl.BlockSpec(s, lambda i: (0, 0))
    return pl.pallas_call(
        _conv_body,
        grid=grid,
        in_specs=[eb(NS), eb(NS), eb(NS), eb(16),
                  wb((3 * NS, 3 * NS)), wb((1, 3 * NS)),
                  wb((3 * NS, NS * SH)), wb((1, NS * SH)),
                  wb((NS * SH, NS))],
        out_specs=eb(NS),
        out_shape=jax.ShapeDtypeStruct((e_count, NS), jnp.float32),
        compiler_params=pltpu.CompilerParams(
            dimension_semantics=("parallel",)),
    )(e, hsrc, hdst, dvec, Wc1.T, bc1.reshape(1, -1), Wc2.T,
      bc2.reshape(1, -1), sel)


def _update_body(h_ref, acc_ref, o_ref):
    acc = acc_ref[...]
    deg = jnp.maximum(acc[:, NS:NS + 1], 1.0)
    o_ref[...] = h_ref[...] + acc[:, :NS] / deg


def _h_update(h, acc):
    n = h.shape[0]
    return pl.pallas_call(
        _update_body,
        out_shape=jax.ShapeDtypeStruct((n, NS), jnp.float32),
    )(h, acc)


def _gather_rows(table, idx):
    return jnp.take(table, idx, axis=0)


def _scatter_acc(msg, dst, n, with_ones):
    ones = jnp.ones((msg.shape[0], 1), jnp.float32)
    pad = jnp.concatenate([msg, ones, jnp.zeros((msg.shape[0], 15), jnp.float32)],
                          axis=1)
    return jax.ops.segment_sum(pad, dst, num_segments=n)


def kernel(x, pos, node_sigma_emb, edge_attr, W_node, b_node, We1, be1, We2,
           be2, Wc1_0, bc1_0, Wc2_0, bc2_0, Wc1_1, bc1_1, Wc2_1, bc2_1,
           edge_index):
    n = x.shape[0]
    src = edge_index[0]
    dst = edge_index[1]

    h = _node_embed(x, node_sigma_emb, W_node, b_node)

    pos16 = jnp.concatenate([pos, jnp.zeros((n, 13), jnp.float32)], axis=1)
    psrc = _gather_rows(pos16, src)
    pdst = _gather_rows(pos16, dst)
    sigsrc = _gather_rows(node_sigma_emb, src)
    ea8 = jnp.concatenate(
        [edge_attr, jnp.zeros((edge_attr.shape[0], 3), jnp.float32)], axis=1)

    e, dvec = _edge_prep(ea8, psrc, pdst, sigsrc, We1, be1, We2, be2)

    ii = jnp.arange(NS * SH) // SH
    sel = (ii[:, None] == jnp.arange(NS)[None, :]).astype(jnp.float32)

    for (Wc1, bc1, Wc2, bc2) in ((Wc1_0, bc1_0, Wc2_0, bc2_0),
                                 (Wc1_1, bc1_1, Wc2_1, bc2_1)):
        hsrc = _gather_rows(h, src)
        hdst = _gather_rows(h, dst)
        msg = _conv_layer(e, hsrc, hdst, dvec, Wc1, bc1, Wc2, bc2, sel)
        acc = _scatter_acc(msg, dst, n, True)
        h = _h_update(h, acc)
    return h


# speedup vs baseline: 3.6668x; 2.6884x over previous
import functools
import math

import jax
import jax.numpy as jnp
from jax import lax
from jax.experimental import pallas as pl
from jax.experimental.pallas import tpu as pltpu
from jax.experimental.pallas import tpu_sc as plsc

NS = 48
SH = 9
LW = 128

EDGE_BLK = 1000

_MM_PREC = lax.Precision.DEFAULT


def _dot(a, b):
    return jnp.dot(a, b, precision=_MM_PREC, preferred_element_type=jnp.float32)


def _node_embed_body(x_ref, sig_ref, wx_ref, ws_ref, b_ref, o_ref):
    acc = _dot(x_ref[...], wx_ref[...]) + _dot(sig_ref[...], ws_ref[...])
    h = jnp.maximum(acc + b_ref[...], 0.0)
    o_ref[...] = jnp.concatenate(
        [h, jnp.zeros((h.shape[0], LW - NS), jnp.float32)], axis=1)


def _node_embed(x, sig, W_node, b_node):
    n = x.shape[0]
    return pl.pallas_call(
        _node_embed_body,
        out_shape=jax.ShapeDtypeStruct((n, LW), jnp.float32),
    )(x, sig, W_node[:, :16].T, W_node[:, 16:].T, b_node.reshape(1, NS))


def _edge_prep_body(ea_ref, ps_ref, pd_ref, w1a_ref, w1b_ref, w1c_ref,
                    b1_ref, w2_ref, b2_ref, e_ref, dv_ref):
    ps = ps_ref[...]
    pd = pd_ref[...]
    vec = pd[:, 0:3] - ps[:, 0:3]
    d2 = jnp.sum(vec * vec, axis=1, keepdims=True)
    d = jnp.sqrt(d2)
    offs = lax.broadcasted_iota(jnp.int32, (1, 32), 1).astype(jnp.float32) \
        * (5.0 / 31.0)
    coeff = -0.5 / (5.0 / 31.0) ** 2
    smear = jnp.exp(coeff * (d - offs) ** 2)
    acc = (_dot(ea_ref[...], w1a_ref[...])
           + _dot(ps[:, 3:35], w1b_ref[...])
           + _dot(smear, w1c_ref[...]))
    h1 = jnp.maximum(acc + b1_ref[...], 0.0)
    e_ref[...] = _dot(h1, w2_ref[...]) + b2_ref[...]
    dvec = vec / jnp.maximum(d, 1e-8)
    blk = dvec.shape[0]
    dv_ref[...] = jnp.concatenate(
        [dvec, jnp.zeros((blk, 13), jnp.float32)], axis=1)


def _edge_prep(edge_attr8, gsrc, gdst, We1, be1, We2, be2):
    e_count = edge_attr8.shape[0]
    grid = (e_count // EDGE_BLK,)
    eb = lambda w: pl.BlockSpec((EDGE_BLK, w), lambda i: (i, 0))
    wb = lambda s: pl.BlockSpec(s, lambda i: (0, 0))
    return pl.pallas_call(
        _edge_prep_body,
        grid=grid,
        in_specs=[eb(8), eb(LW), eb(LW),
                  wb((8, NS)), wb((32, NS)), wb((32, NS)), wb((1, NS)),
                  wb((NS, NS)), wb((1, NS))],
        out_specs=[eb(NS), eb(16)],
        out_shape=[jax.ShapeDtypeStruct((e_count, NS), jnp.float32),
                   jax.ShapeDtypeStruct((e_count, 16), jnp.float32)],
        compiler_params=pltpu.CompilerParams(
            dimension_semantics=("parallel",)),
    )(edge_attr8, gsrc, gdst,
      jnp.concatenate([We1[:, :5].T, jnp.zeros((3, NS), jnp.float32)], axis=0),
      We1[:, 5:37].T, We1[:, 37:69].T, be1.reshape(1, NS),
      We2.T, be2.reshape(1, NS))


def _conv_body(e_ref, hs_ref, hd_ref, dv_ref, w1_ref, b1_ref, w2_ref, b2_ref,
               s_ref, o_ref):
    hs = hs_ref[:, 0:NS]
    feat = jnp.concatenate([e_ref[...], hs, hd_ref[:, 0:NS]], axis=1)
    a = jnp.maximum(_dot(feat, w1_ref[...]) + b1_ref[...], 0.0)
    w2 = _dot(a, w2_ref[...]) + b2_ref[...]
    dv = dv_ref[...]
    dx, dy, dz = dv[:, 0:1], dv[:, 1:2], dv[:, 2:3]
    s3 = math.sqrt(3.0)
    s15 = math.sqrt(15.0)
    s5 = math.sqrt(5.0)
    shv = [jnp.ones_like(dx), s3 * dx, s3 * dy, s3 * dz,
           s15 * dx * dy, s15 * dy * dz,
           (s5 / 2.0) * (3.0 * dz * dz - 1.0), s15 * dx * dz,
           (s15 / 2.0) * (dx * dx - dy * dy)]
    blk = w2.shape[0]
    lane = lax.broadcasted_iota(jnp.int32, (blk, NS * SH), 1)
    jmod = lane - (lane // SH) * SH
    shexp = jnp.zeros_like(w2)
    for j in range(SH):
        shexp = jnp.where(jmod == j, shv[j], shexp)
    t = _dot(w2 * shexp, s_ref[...])
    o_ref[...] = jnp.concatenate(
        [hs * t, jnp.ones((blk, 1), jnp.float32),
         jnp.zeros((blk, LW - NS - 1), jnp.float32)], axis=1)


def _conv_layer(e, hsrc, hdst, dvec, Wc1, bc1, Wc2, bc2, sel):
    e_count = e.shape[0]
    grid = (e_count // EDGE_BLK,)
    eb = lambda w: pl.BlockSpec((EDGE_BLK, w), lambda i: (i, 0))
    wb = lambda s: pl.BlockSpec(s, lambda i: (0, 0))
    return pl.pallas_call(
        _conv_body,
        grid=grid,
        in_specs=[eb(NS), eb(LW), eb(LW), eb(16),
                  wb((3 * NS, 3 * NS)), wb((1, 3 * NS)),
                  wb((3 * NS, NS * SH)), wb((1, NS * SH)),
                  wb((NS * SH, NS))],
        out_specs=eb(LW),
        out_shape=jax.ShapeDtypeStruct((e_count, LW), jnp.float32),
        compiler_params=pltpu.CompilerParams(
            dimension_semantics=("parallel",)),
    )(e, hsrc, hdst, dvec, Wc1.T, bc1.reshape(1, -1), Wc2.T,
      bc2.reshape(1, -1), sel)


def _update_body(final, h_ref, acc_ref, o_ref):
    nn = h_ref.shape[0]
    agg = acc_ref[0, 0:nn, 0:NS] + acc_ref[1, 0:nn, 0:NS]
    deg = jnp.maximum(
        acc_ref[0, 0:nn, NS:NS + 1] + acc_ref[1, 0:nn, NS:NS + 1], 1.0)
    hn = h_ref[:, 0:NS] + agg / deg
    if final:
        o_ref[...] = hn
    else:
        o_ref[...] = jnp.concatenate(
            [hn, jnp.zeros((hn.shape[0], LW - NS), jnp.float32)], axis=1)


def _h_update(h, acc, final):
    n = h.shape[0]
    w = NS if final else LW
    return pl.pallas_call(
        functools.partial(_update_body, final),
        out_shape=jax.ShapeDtypeStruct((n, w), jnp.float32),
    )(h, acc)


_SC_CORES = 2
_SC_SUBCORES = 16
_SC_WORKERS = _SC_CORES * _SC_SUBCORES
_STREAM = 80


def _sc_mesh():
    return plsc.VectorSubcoreMesh(core_axis_name="c", subcore_axis_name="s")


_N_PAD = 10240


@jax.jit
def _sc_gather2(tab_a, idx_a3, tab_b, idx_b3):
    _, rows_w, _ = idx_a3.shape
    e_count = _SC_WORKERS * rows_w * _STREAM

    @functools.partial(
        pl.kernel,
        out_type=[jax.ShapeDtypeStruct((e_count, LW), jnp.float32),
                  jax.ShapeDtypeStruct((e_count, LW), jnp.float32)],
        mesh=_sc_mesh(),
        scratch_types=[pltpu.VMEM((rows_w, _STREAM), jnp.int32),
                       pltpu.VMEM((_STREAM, LW), jnp.float32),
                       pltpu.VMEM((rows_w, _STREAM), jnp.int32),
                       pltpu.VMEM((_STREAM, LW), jnp.float32),
                       pltpu.SemaphoreType.DMA,
                       pltpu.SemaphoreType.DMA],
    )
    def k(ta_hbm, ia_hbm, tb_hbm, ib_hbm, oa_hbm, ob_hbm,
          ia_v, ra_v, ib_v, rb_v, sa, sb):
        wid = lax.axis_index("s") * _SC_CORES + lax.axis_index("c")
        base = wid * rows_w * _STREAM
        pltpu.sync_copy(ia_hbm.at[wid], ia_v)
        pltpu.sync_copy(ib_hbm.at[wid], ib_v)

        @pl.loop(0, rows_w)
        def _(j):
            off = base + j * _STREAM
            ca = pltpu.async_copy(ta_hbm.at[ia_v.at[j]], ra_v, sa)
            cb = pltpu.async_copy(tb_hbm.at[ib_v.at[j]], rb_v, sb)
            ca.wait()
            pltpu.sync_copy(ra_v, oa_hbm.at[pl.ds(off, _STREAM)])
            cb.wait()
            pltpu.sync_copy(rb_v, ob_hbm.at[pl.ds(off, _STREAM)])

    return k(tab_a, idx_a3, tab_b, idx_b3)


@jax.jit
def _sc_scatter(msg, dst3, zeros_lw):
    _, rows_w, _ = dst3.shape
    stripe = _N_PAD // _SC_SUBCORES

    @functools.partial(
        pl.kernel,
        out_type=jax.ShapeDtypeStruct((_SC_CORES, _N_PAD, LW), jnp.float32),
        mesh=_sc_mesh(),
        scratch_types=[pltpu.VMEM((rows_w, _STREAM), jnp.int32),
                       pltpu.VMEM((_STREAM, LW), jnp.float32),
                       pltpu.VMEM_SHARED((_N_PAD, LW), jnp.float32)],
    )
    def k(msg_hbm, dst_hbm, z_hbm, acc_hbm, idx_v, msg_v, acc_sh):
        c = lax.axis_index("c")
        s = lax.axis_index("s")
        wid = s * _SC_CORES + c
        base = wid * rows_w * _STREAM
        pltpu.sync_copy(z_hbm.at[pl.ds(s * stripe, stripe)],
                        acc_sh.at[pl.ds(s * stripe, stripe)])
        pltpu.sync_copy(dst_hbm.at[wid], idx_v)
        plsc.subcore_barrier()

        @pl.loop(0, rows_w)
        def _(j):
            off = base + j * _STREAM
            pltpu.sync_copy(msg_hbm.at[pl.ds(off, _STREAM)], msg_v)
            pltpu.sync_copy(msg_v, acc_sh.at[idx_v.at[j]], add=True)

        plsc.subcore_barrier()
        pltpu.sync_copy(acc_sh.at[pl.ds(s * stripe, stripe)],
                        acc_hbm.at[c, pl.ds(s * stripe, stripe)])

    return k(msg, dst3, zeros_lw)


def kernel(x, pos, node_sigma_emb, edge_attr, W_node, b_node, We1, be1, We2,
           be2, Wc1_0, bc1_0, Wc2_0, bc2_0, Wc1_1, bc1_1, Wc2_1, bc2_1,
           edge_index):
    n = x.shape[0]
    e_count = edge_index.shape[1]
    src3 = edge_index[0].reshape(_SC_WORKERS, -1, _STREAM)
    dst3 = edge_index[1].reshape(_SC_WORKERS, -1, _STREAM)

    h = _node_embed(x, node_sigma_emb, W_node, b_node)

    geo = jnp.concatenate([pos, node_sigma_emb,
                           jnp.zeros((n, LW - 35), jnp.float32)], axis=1)
    ea8 = jnp.concatenate(
        [edge_attr, jnp.zeros((e_count, 3), jnp.float32)], axis=1)

    gsrc, gdst = _sc_gather2(geo, src3, geo, dst3)
    e, dvec = _edge_prep(ea8, gsrc, gdst, We1, be1, We2, be2)

    ii = jnp.arange(NS * SH) // SH
    sel = (ii[:, None] == jnp.arange(NS)[None, :]).astype(jnp.float32)

    zeros_lw = jnp.zeros((_N_PAD, LW), jnp.float32)

    for li, (Wc1, bc1, Wc2, bc2) in enumerate(
            ((Wc1_0, bc1_0, Wc2_0, bc2_0), (Wc1_1, bc1_1, Wc2_1, bc2_1))):
        hsrc, hdst = _sc_gather2(h, src3, h, dst3)
        msg = _conv_layer(e, hsrc, hdst, dvec, Wc1, bc1, Wc2, bc2, sel)
        acc = _sc_scatter(msg, dst3, zeros_lw)
        h = _h_update(h, acc, final=(li == 1))
    return h
